# baseline (speedup 1.0000x reference)
import jax
import jax.numpy as jnp
from jax import lax
from jax.experimental import pallas as pl
from jax.experimental.pallas import tpu as pltpu
from jax.experimental.pallas import tpu_sc as plsc

_B, _S, _D = 4, 8192, 1024
_NC, _NS = 2, 16
_K = 8
_NVEC = _K * _D // 16
_XROWS = _B * _S
_EROWS = _S
_SR0, _SR1 = 304, 208
_S0 = _NS * _SR0


def _sc_body(x_hbm, e_hbm, o_hbm,
             xb0, xb1, xb2, xb3, ob0, ob1, ob2, ob3, eb0, eb1,
             slx0, slx1, slx2, slx3, sst0, sst1, sst2, sst3, sle0, sle1):
    cid = lax.axis_index("c")
    sid = lax.axis_index("s")
    is0 = cid == 0
    sbase = jnp.where(is0, sid * _SR0, _S0 + sid * _SR1)
    nch = jnp.where(is0, _SR0 // _K, _SR1 // _K)

    xb = (xb0, xb1, xb2, xb3)
    ob = (ob0, ob1, ob2, ob3)
    eb = (eb0, eb1)
    slx = (slx0, slx1, slx2, slx3)
    sst = (sst0, sst1, sst2, sst3)
    sle = (sle0, sle1)

    def xrow(ci, b):
        return jnp.minimum(b * _S + sbase + ci * _K, _XROWS - _K)

    def erow(ci):
        return jnp.minimum(sbase + ci * _K, _EROWS - _K)

    def issue_xload(ci, b):
        pltpu.async_copy(x_hbm.at[pl.ds(xrow(ci, b), _K)], xb[b], slx[b])

    def wait_xload(ci, b):
        pltpu.make_async_copy(x_hbm.at[pl.ds(xrow(ci, b), _K)], xb[b], slx[b]).wait()

    def issue_eload(ci, pe):
        pltpu.async_copy(e_hbm.at[pl.ds(erow(ci), _K)], eb[pe], sle[pe])

    def wait_eload(ci, pe):
        pltpu.make_async_copy(e_hbm.at[pl.ds(erow(ci), _K)], eb[pe], sle[pe]).wait()

    def compute(b, pe):
        xr, er, orr = xb[b], eb[pe], ob[b]

        @plsc.parallel_loop(0, _NVEC, unroll=8)
        def _vec(j):
            i = j >> 6
            c = pl.multiple_of((j & 63) << 4, 16)
            orr[i, pl.ds(c, 16)] = xr[i, pl.ds(c, 16)] + er[i, pl.ds(c, 16)]

    def issue_store(ci, b):
        pltpu.async_copy(ob[b], o_hbm.at[pl.ds(xrow(ci, b), _K)], sst[b])

    def wait_store(ci, b):
        pltpu.make_async_copy(ob[b], o_hbm.at[pl.ds(xrow(ci, b), _K)], sst[b]).wait()

    def do_chunk(ci, pe, first):
        for b in range(_B):
            wait_xload(ci, b)
            if b == 0:
                wait_eload(ci, pe)
            if not first:
                wait_store(ci - 1, b)
            compute(b, pe)
            issue_store(ci, b)
            issue_xload(ci + 1, b)
        issue_eload(ci + 2, pe)

    issue_eload(0, 0)
    issue_eload(1, 1)
    for b in range(_B):
        issue_xload(0, b)
    do_chunk(0, 0, True)
    do_chunk(1, 1, False)

    @pl.loop(1, nch // 2)
    def _pipe(i0):
        do_chunk(i0 * 2, 0, False)
        do_chunk(i0 * 2 + 1, 1, False)

    for b in range(_B):
        wait_store(nch - 1, b)
        wait_xload(nch, b)
    wait_eload(nch, 0)
    wait_eload(nch + 1, 1)


def kernel(x, emb_weight):
    mesh = plsc.VectorSubcoreMesh(core_axis_name="c", subcore_axis_name="s")
    k = pl.kernel(
        _sc_body,
        out_type=jax.ShapeDtypeStruct((_XROWS, _D), jnp.float32),
        mesh=mesh,
        compiler_params=pltpu.CompilerParams(use_tc_tiling_on_sc=True),
        scratch_types=(
            [pltpu.VMEM((_K, _D), jnp.float32) for _ in range(10)]
            + [pltpu.SemaphoreType.DMA for _ in range(10)]
        ),
    )
    out = k(x.reshape(_XROWS, _D), emb_weight)
    return out.reshape(x.shape)
